# baseline (speedup 1.0000x reference)
import functools

import jax
import jax.numpy as jnp
import numpy as np
from jax import lax
from jax.experimental import pallas as pl
from jax.experimental.pallas import tpu as pltpu
from jax.experimental.pallas import tpu_sc as plsc

N = 10000
E = 320000
D = 128
NC = 2
NS = 16
NW = NC * NS
L = 16
NPAD = 10240
EPT = E // NW
CH = 64
BCH = 16
NBLK = 10
NCHUNK = NBLK * BCH
EPTP = NCHUNK * CH
ETOT = NW * EPTP
CPT = D // NW
BK2 = 2048
NBK2 = ETOT // BK2
NPR2 = NBK2 // 2
RPT = NPAD // NS

ROWB = 512
GRID = NPAD // ROWB


def _tc1_body(x_ref, w_ref, as_ref, ad_ref, h_ref, aso_ref, ado_ref):
    h = jnp.dot(x_ref[...], w_ref[...], preferred_element_type=jnp.float32)
    h_ref[...] = h
    aso_ref[...] = jnp.sum(h * as_ref[...], axis=1, keepdims=True)
    ado_ref[...] = jnp.sum(h * ad_ref[...], axis=1, keepdims=True)


def _tc1(xp, W, asv, adv):
    return pl.pallas_call(
        _tc1_body,
        grid=(GRID,),
        in_specs=[
            pl.BlockSpec((ROWB, D), lambda i: (i, 0)),
            pl.BlockSpec((D, D), lambda i: (0, 0)),
            pl.BlockSpec((1, D), lambda i: (0, 0)),
            pl.BlockSpec((1, D), lambda i: (0, 0)),
        ],
        out_specs=[
            pl.BlockSpec((ROWB, D), lambda i: (i, 0)),
            pl.BlockSpec((ROWB, 1), lambda i: (i, 0)),
            pl.BlockSpec((ROWB, 1), lambda i: (i, 0)),
        ],
        out_shape=[
            jax.ShapeDtypeStruct((NPAD, D), jnp.float32),
            jax.ShapeDtypeStruct((NPAD, 1), jnp.float32),
            jax.ShapeDtypeStruct((NPAD, 1), jnp.float32),
        ],
    )(xp, W, asv, adv)


def _tc2_body(p_ref, d0_ref, d1_ref, b_ref, w_ref, as_ref, ad_ref,
              h_ref, aso_ref, ado_ref):
    den = d0_ref[...] + d1_ref[...]
    h1 = p_ref[...] / (den + 1e-16) + b_ref[...]
    h1 = jnp.where(h1 > 0, h1, jnp.exp(jnp.minimum(h1, 0.0)) - 1.0)
    h2 = jnp.dot(h1, w_ref[...], preferred_element_type=jnp.float32)
    h_ref[...] = h2
    aso_ref[...] = jnp.sum(h2 * as_ref[...], axis=1, keepdims=True)
    ado_ref[...] = jnp.sum(h2 * ad_ref[...], axis=1, keepdims=True)


def _tc2(p, d0, d1, b, W, asv, adv):
    return pl.pallas_call(
        _tc2_body,
        grid=(GRID,),
        in_specs=[
            pl.BlockSpec((ROWB, D), lambda i: (i, 0)),
            pl.BlockSpec((ROWB, 1), lambda i: (i, 0)),
            pl.BlockSpec((ROWB, 1), lambda i: (i, 0)),
            pl.BlockSpec((1, D), lambda i: (0, 0)),
            pl.BlockSpec((D, D), lambda i: (0, 0)),
            pl.BlockSpec((1, D), lambda i: (0, 0)),
            pl.BlockSpec((1, D), lambda i: (0, 0)),
        ],
        out_specs=[
            pl.BlockSpec((ROWB, D), lambda i: (i, 0)),
            pl.BlockSpec((ROWB, 1), lambda i: (i, 0)),
            pl.BlockSpec((ROWB, 1), lambda i: (i, 0)),
        ],
        out_shape=[
            jax.ShapeDtypeStruct((NPAD, D), jnp.float32),
            jax.ShapeDtypeStruct((NPAD, 1), jnp.float32),
            jax.ShapeDtypeStruct((NPAD, 1), jnp.float32),
        ],
    )(p, d0, d1, b, W, asv, adv)


def _tc3_body(p_ref, d0_ref, d1_ref, b_ref, o_ref):
    den = d0_ref[...] + d1_ref[...]
    o_ref[...] = p_ref[...] / (den + 1e-16) + b_ref[...]


def _tc3(p, d0, d1, b):
    return pl.pallas_call(
        _tc3_body,
        grid=(GRID,),
        in_specs=[
            pl.BlockSpec((ROWB, D), lambda i: (i, 0)),
            pl.BlockSpec((ROWB, 1), lambda i: (i, 0)),
            pl.BlockSpec((ROWB, 1), lambda i: (i, 0)),
            pl.BlockSpec((1, D), lambda i: (0, 0)),
        ],
        out_specs=pl.BlockSpec((ROWB, D), lambda i: (i, 0)),
        out_shape=jax.ShapeDtypeStruct((NPAD, D), jnp.float32),
    )(p, d0, d1, b)


def _sc_weights(src3, dst3, a_s, a_d):
    mesh = plsc.VectorSubcoreMesh(core_axis_name="c", subcore_axis_name="s")

    @functools.partial(
        pl.kernel,
        mesh=mesh,
        compiler_params=pltpu.CompilerParams(needs_layout_passes=False),
        out_type=[
            jax.ShapeDtypeStruct((NW, NCHUNK, CH), jnp.float32),
            jax.ShapeDtypeStruct((NPAD,), jnp.float32),
            jax.ShapeDtypeStruct((NPAD,), jnp.float32),
        ],
        scratch_types=[
            pltpu.VMEM((NPAD,), jnp.float32),
            pltpu.VMEM((NPAD,), jnp.float32),
            pltpu.VMEM((BCH, CH), jnp.int32),
            pltpu.VMEM((BCH, CH), jnp.int32),
            pltpu.VMEM((BCH, CH), jnp.float32),
            pltpu.VMEM((RPT,), jnp.float32),
            pltpu.VMEM_SHARED((NPAD,), jnp.float32),
            pltpu.SemaphoreType.DMA,
        ],
    )
    def ska(src_h, dst_h, as_h, ad_h, p_h, d0_h, d1_h,
            as_t, ad_t, s_blk, d_blk, p_blk, zden, den_acc, dsem):
        c = lax.axis_index("c")
        s = lax.axis_index("s")
        w = c * NS + s
        base = s * RPT

        pltpu.sync_copy(as_h, as_t)
        pltpu.sync_copy(ad_h, ad_t)

        zv = jnp.zeros((L,), jnp.float32)

        def zden_body(i, _):
            zden[pl.ds(i * L, L)] = zv
            return 0
        lax.fori_loop(0, RPT // L, zden_body, 0)
        pltpu.sync_copy(zden, den_acc.at[pl.ds(base, RPT)])
        plsc.subcore_barrier()

        def blk_body(bk, _):
            pltpu.sync_copy(src_h.at[w, pl.ds(bk * BCH, BCH)], s_blk)
            pltpu.sync_copy(dst_h.at[w, pl.ds(bk * BCH, BCH)], d_blk)

            def ch_body(k, _):
                for v in range(CH // L):
                    sl = pl.ds(v * L, L)
                    e = (plsc.load_gather(as_t, [s_blk[k, sl]])
                         + plsc.load_gather(ad_t, [d_blk[k, sl]]))
                    e = jnp.maximum(e, e * 0.2)
                    p_blk[k, sl] = jnp.exp(e)
                pltpu.async_copy(p_blk.at[k], den_acc.at[d_blk.at[k]], dsem,
                                 add=True)
                return 0
            lax.fori_loop(0, BCH, ch_body, 0)

            def drain_body(k, _):
                pltpu.make_async_copy(p_blk.at[k], den_acc.at[d_blk.at[k]],
                                      dsem).wait()
                return 0
            lax.fori_loop(0, BCH, drain_body, 0)
            pltpu.sync_copy(p_blk, p_h.at[w, pl.ds(bk * BCH, BCH)])
            return 0
        lax.fori_loop(0, NBLK, blk_body, 0)

        plsc.subcore_barrier()

        @pl.when(c == 0)
        def _():
            pltpu.sync_copy(den_acc.at[pl.ds(base, RPT)],
                            d0_h.at[pl.ds(base, RPT)])

        @pl.when(c == 1)
        def _():
            pltpu.sync_copy(den_acc.at[pl.ds(base, RPT)],
                            d1_h.at[pl.ds(base, RPT)])

    return ska(src3, dst3, a_s, a_d)


def _sc_aggregate(srcf, dstf, pf, hT):
    mesh = plsc.VectorSubcoreMesh(core_axis_name="c", subcore_axis_name="s")

    @functools.partial(
        pl.kernel,
        mesh=mesh,
        compiler_params=pltpu.CompilerParams(needs_layout_passes=False),
        out_type=jax.ShapeDtypeStruct((NW, CPT, NPAD), jnp.float32),
        scratch_types=[
            pltpu.VMEM((CPT, NPAD), jnp.float32),
            pltpu.VMEM((CPT, NPAD), jnp.float32),
            pltpu.VMEM((2, BK2), jnp.int32),
            pltpu.VMEM((2, BK2), jnp.int32),
            pltpu.VMEM((2, BK2), jnp.float32),
            pltpu.SemaphoreType.DMA,
        ],
    )
    def skb(src_h, dst_h, p_h, hT_h, out_h,
            h_t, acc, s_st, d_st, p_st, stsem):
        c = lax.axis_index("c")
        s = lax.axis_index("s")
        w = c * NS + s

        pltpu.sync_copy(hT_h.at[pl.ds(w * CPT, CPT)], h_t)

        zv = jnp.zeros((L,), jnp.float32)

        def zacc_body(i, _):
            for j in range(CPT):
                acc[j, pl.ds(i * L, L)] = zv
            return 0
        lax.fori_loop(0, NPAD // L, zacc_body, 0)

        jvs = [jnp.full((L,), j, jnp.int32) for j in range(CPT)]

        def stage(bk, db):
            pltpu.async_copy(src_h.at[bk], s_st.at[db], stsem)
            pltpu.async_copy(dst_h.at[bk], d_st.at[db], stsem)
            pltpu.async_copy(p_h.at[bk], p_st.at[db], stsem)

        def stage_wait(bk, db):
            pltpu.make_async_copy(src_h.at[bk], s_st.at[db], stsem).wait()
            pltpu.make_async_copy(dst_h.at[bk], d_st.at[db], stsem).wait()
            pltpu.make_async_copy(p_h.at[bk], p_st.at[db], stsem).wait()

        stage(0, 0)

        def pair_body(pr, _):
            for db in range(2):
                bk = pr * 2 + db
                stage_wait(bk, db)
                if db == 0:
                    stage(bk + 1, 1)
                else:
                    @pl.when(pr < NPR2 - 1)
                    def _():
                        stage(bk + 1, 0)

                def vec_body(v, _):
                    sl = pl.ds(v * L, L)
                    sv = s_st[db, sl]
                    dv = d_st[db, sl]
                    pv = p_st[db, sl]
                    hvs = [plsc.load_gather(h_t, [jvs[j], sv])
                           for j in range(CPT)]
                    for j in range(CPT):
                        plsc.addupdate_scatter(acc, [jvs[j], dv], hvs[j] * pv)
                    return 0
                lax.fori_loop(0, BK2 // L, vec_body, 0, unroll=8)
            return 0
        lax.fori_loop(0, NPR2, pair_body, 0)

        pltpu.sync_copy(acc, out_h.at[w])

    return skb(srcf, dstf, pf, hT)


def _sc_layer(src3, dst3, srcf, dstf, h, a_s, a_d):
    p3, d0, d1 = _sc_weights(src3, dst3, a_s, a_d)
    pf = p3.reshape(NBK2, BK2)
    outB = _sc_aggregate(srcf, dstf, pf, h.T)
    part = outB.reshape(D, NPAD).T
    return part, d0, d1


def kernel(x, edge_index, W1, att_src1, att_dst1, b1, W2, att_src2, att_dst2, b2):
    f32 = jnp.float32
    src = edge_index[0].reshape(NW, EPT)
    dst = edge_index[1].reshape(NW, EPT)
    pad_s = jnp.zeros((NW, EPTP - EPT), jnp.int32)
    pad_d = jnp.full((NW, EPTP - EPT), NPAD - 1, jnp.int32)
    src3 = jnp.concatenate([src, pad_s], axis=1).reshape(NW, NCHUNK, CH)
    dst3 = jnp.concatenate([dst, pad_d], axis=1).reshape(NW, NCHUNK, CH)
    srcf = src3.reshape(NBK2, BK2)
    dstf = dst3.reshape(NBK2, BK2)

    xp = jnp.zeros((NPAD, D), f32).at[:N].set(x)

    h1, a1s, a1d = _tc1(xp, W1, att_src1.reshape(1, D), att_dst1.reshape(1, D))
    part1, d0, d1 = _sc_layer(src3, dst3, srcf, dstf, h1,
                              a1s.reshape(NPAD), a1d.reshape(NPAD))
    h2, a2s, a2d = _tc2(part1, d0.reshape(NPAD, 1), d1.reshape(NPAD, 1),
                        b1.reshape(1, D), W2,
                        att_src2.reshape(1, D), att_dst2.reshape(1, D))
    part2, e0, e1 = _sc_layer(src3, dst3, srcf, dstf, h2,
                              a2s.reshape(NPAD), a2d.reshape(NPAD))
    out = _tc3(part2, e0.reshape(NPAD, 1), e1.reshape(NPAD, 1),
               b2.reshape(1, D))
    return out[:N]


# speedup vs baseline: 1.1187x; 1.1187x over previous
import functools

import jax
import jax.numpy as jnp
import numpy as np
from jax import lax
from jax.experimental import pallas as pl
from jax.experimental.pallas import tpu as pltpu
from jax.experimental.pallas import tpu_sc as plsc

N = 10000
E = 320000
D = 128
NC = 2
NS = 16
NW = NC * NS
L = 16
NPAD = 10240
EPT = E // NW
CH = 64
BCH = 16
NBLK = 10
NCHUNK = NBLK * BCH
EPTP = NCHUNK * CH
ETOT = NW * EPTP
CPT = D // NW
BK2 = 2048
NBK2 = ETOT // BK2
NPR2 = NBK2 // 2
RPT = NPAD // NS

ROWB = 512
GRID = NPAD // ROWB


def _tc1_body(x_ref, w_ref, as_ref, ad_ref, h_ref, aso_ref, ado_ref):
    h = jnp.dot(x_ref[...], w_ref[...], preferred_element_type=jnp.float32)
    h_ref[...] = h
    aso_ref[...] = jnp.sum(h * as_ref[...], axis=1, keepdims=True)
    ado_ref[...] = jnp.sum(h * ad_ref[...], axis=1, keepdims=True)


def _tc1(xp, W, asv, adv):
    return pl.pallas_call(
        _tc1_body,
        grid=(GRID,),
        in_specs=[
            pl.BlockSpec((ROWB, D), lambda i: (i, 0)),
            pl.BlockSpec((D, D), lambda i: (0, 0)),
            pl.BlockSpec((1, D), lambda i: (0, 0)),
            pl.BlockSpec((1, D), lambda i: (0, 0)),
        ],
        out_specs=[
            pl.BlockSpec((ROWB, D), lambda i: (i, 0)),
            pl.BlockSpec((ROWB, 1), lambda i: (i, 0)),
            pl.BlockSpec((ROWB, 1), lambda i: (i, 0)),
        ],
        out_shape=[
            jax.ShapeDtypeStruct((NPAD, D), jnp.float32),
            jax.ShapeDtypeStruct((NPAD, 1), jnp.float32),
            jax.ShapeDtypeStruct((NPAD, 1), jnp.float32),
        ],
    )(xp, W, asv, adv)


def _tc2_body(p_ref, d0_ref, d1_ref, b_ref, w_ref, as_ref, ad_ref,
              h_ref, aso_ref, ado_ref):
    den = d0_ref[...] + d1_ref[...]
    h1 = p_ref[...] / (den + 1e-16) + b_ref[...]
    h1 = jnp.where(h1 > 0, h1, jnp.exp(jnp.minimum(h1, 0.0)) - 1.0)
    h2 = jnp.dot(h1, w_ref[...], preferred_element_type=jnp.float32)
    h_ref[...] = h2
    aso_ref[...] = jnp.sum(h2 * as_ref[...], axis=1, keepdims=True)
    ado_ref[...] = jnp.sum(h2 * ad_ref[...], axis=1, keepdims=True)


def _tc2(p, d0, d1, b, W, asv, adv):
    return pl.pallas_call(
        _tc2_body,
        grid=(GRID,),
        in_specs=[
            pl.BlockSpec((ROWB, D), lambda i: (i, 0)),
            pl.BlockSpec((ROWB, 1), lambda i: (i, 0)),
            pl.BlockSpec((ROWB, 1), lambda i: (i, 0)),
            pl.BlockSpec((1, D), lambda i: (0, 0)),
            pl.BlockSpec((D, D), lambda i: (0, 0)),
            pl.BlockSpec((1, D), lambda i: (0, 0)),
            pl.BlockSpec((1, D), lambda i: (0, 0)),
        ],
        out_specs=[
            pl.BlockSpec((ROWB, D), lambda i: (i, 0)),
            pl.BlockSpec((ROWB, 1), lambda i: (i, 0)),
            pl.BlockSpec((ROWB, 1), lambda i: (i, 0)),
        ],
        out_shape=[
            jax.ShapeDtypeStruct((NPAD, D), jnp.float32),
            jax.ShapeDtypeStruct((NPAD, 1), jnp.float32),
            jax.ShapeDtypeStruct((NPAD, 1), jnp.float32),
        ],
    )(p, d0, d1, b, W, asv, adv)


def _tc3_body(p_ref, d0_ref, d1_ref, b_ref, o_ref):
    den = d0_ref[...] + d1_ref[...]
    o_ref[...] = p_ref[...] / (den + 1e-16) + b_ref[...]


def _tc3(p, d0, d1, b):
    return pl.pallas_call(
        _tc3_body,
        grid=(GRID,),
        in_specs=[
            pl.BlockSpec((ROWB, D), lambda i: (i, 0)),
            pl.BlockSpec((ROWB, 1), lambda i: (i, 0)),
            pl.BlockSpec((ROWB, 1), lambda i: (i, 0)),
            pl.BlockSpec((1, D), lambda i: (0, 0)),
        ],
        out_specs=pl.BlockSpec((ROWB, D), lambda i: (i, 0)),
        out_shape=jax.ShapeDtypeStruct((NPAD, D), jnp.float32),
    )(p, d0, d1, b)


def _sc_weights(src3, dst3, a_s, a_d):
    mesh = plsc.VectorSubcoreMesh(core_axis_name="c", subcore_axis_name="s")

    @functools.partial(
        pl.kernel,
        mesh=mesh,
        compiler_params=pltpu.CompilerParams(needs_layout_passes=False),
        out_type=[
            jax.ShapeDtypeStruct((NW, NCHUNK, CH), jnp.float32),
            jax.ShapeDtypeStruct((NPAD,), jnp.float32),
            jax.ShapeDtypeStruct((NPAD,), jnp.float32),
        ],
        scratch_types=[
            pltpu.VMEM((NPAD,), jnp.float32),
            pltpu.VMEM((NPAD,), jnp.float32),
            pltpu.VMEM((BCH, CH), jnp.int32),
            pltpu.VMEM((BCH, CH), jnp.int32),
            pltpu.VMEM((BCH, CH), jnp.float32),
            pltpu.VMEM((RPT,), jnp.float32),
            pltpu.VMEM_SHARED((NPAD,), jnp.float32),
            pltpu.SemaphoreType.DMA,
        ],
    )
    def ska(src_h, dst_h, as_h, ad_h, p_h, d0_h, d1_h,
            as_t, ad_t, s_blk, d_blk, p_blk, zden, den_acc, dsem):
        c = lax.axis_index("c")
        s = lax.axis_index("s")
        w = c * NS + s
        base = s * RPT

        pltpu.sync_copy(as_h, as_t)
        pltpu.sync_copy(ad_h, ad_t)

        zv = jnp.zeros((L,), jnp.float32)

        def zden_body(i, _):
            zden[pl.ds(i * L, L)] = zv
            return 0
        lax.fori_loop(0, RPT // L, zden_body, 0)
        pltpu.sync_copy(zden, den_acc.at[pl.ds(base, RPT)])
        plsc.subcore_barrier()

        def blk_body(bk, _):
            pltpu.sync_copy(src_h.at[w, pl.ds(bk * BCH, BCH)], s_blk)
            pltpu.sync_copy(dst_h.at[w, pl.ds(bk * BCH, BCH)], d_blk)

            def ch_body(k, _):
                for v in range(CH // L):
                    sl = pl.ds(v * L, L)
                    e = (plsc.load_gather(as_t, [s_blk[k, sl]])
                         + plsc.load_gather(ad_t, [d_blk[k, sl]]))
                    e = jnp.maximum(e, e * 0.2)
                    p_blk[k, sl] = jnp.exp(e)
                pltpu.async_copy(p_blk.at[k], den_acc.at[d_blk.at[k]], dsem,
                                 add=True)
                return 0
            lax.fori_loop(0, BCH, ch_body, 0)

            def drain_body(k, _):
                pltpu.make_async_copy(p_blk.at[k], den_acc.at[d_blk.at[k]],
                                      dsem).wait()
                return 0
            lax.fori_loop(0, BCH, drain_body, 0)
            pltpu.sync_copy(p_blk, p_h.at[w, pl.ds(bk * BCH, BCH)])
            return 0
        lax.fori_loop(0, NBLK, blk_body, 0)

        plsc.subcore_barrier()

        @pl.when(c == 0)
        def _():
            pltpu.sync_copy(den_acc.at[pl.ds(base, RPT)],
                            d0_h.at[pl.ds(base, RPT)])

        @pl.when(c == 1)
        def _():
            pltpu.sync_copy(den_acc.at[pl.ds(base, RPT)],
                            d1_h.at[pl.ds(base, RPT)])

    return ska(src3, dst3, a_s, a_d)


def _sc_aggregate(srcf, dstf, pf, hT):
    mesh = plsc.VectorSubcoreMesh(core_axis_name="c", subcore_axis_name="s")

    @functools.partial(
        pl.kernel,
        mesh=mesh,
        compiler_params=pltpu.CompilerParams(needs_layout_passes=False),
        out_type=jax.ShapeDtypeStruct((NW, CPT, NPAD), jnp.float32),
        scratch_types=(
            [pltpu.VMEM((NPAD,), jnp.float32) for _ in range(CPT)]
            + [pltpu.VMEM((NPAD,), jnp.float32) for _ in range(CPT)]
            + [
                pltpu.VMEM((2, BK2), jnp.int32),
                pltpu.VMEM((2, BK2), jnp.int32),
                pltpu.VMEM((2, BK2), jnp.float32),
                pltpu.SemaphoreType.DMA,
            ]
        ),
    )
    def skb(src_h, dst_h, p_h, hT_h, out_h,
            ht0, ht1, ht2, ht3, ac0, ac1, ac2, ac3, s_st, d_st, p_st, stsem):
        c = lax.axis_index("c")
        s = lax.axis_index("s")
        w = c * NS + s
        hts = [ht0, ht1, ht2, ht3]
        accs = [ac0, ac1, ac2, ac3]

        for j in range(CPT):
            pltpu.sync_copy(hT_h.at[w * CPT + j], hts[j])

        zv = jnp.zeros((L,), jnp.float32)

        def zacc_body(i, _):
            for j in range(CPT):
                accs[j][pl.ds(i * L, L)] = zv
            return 0
        lax.fori_loop(0, NPAD // L, zacc_body, 0)

        def stage(bk, db):
            pltpu.async_copy(src_h.at[bk], s_st.at[db], stsem)
            pltpu.async_copy(dst_h.at[bk], d_st.at[db], stsem)
            pltpu.async_copy(p_h.at[bk], p_st.at[db], stsem)

        def stage_wait(bk, db):
            pltpu.make_async_copy(src_h.at[bk], s_st.at[db], stsem).wait()
            pltpu.make_async_copy(dst_h.at[bk], d_st.at[db], stsem).wait()
            pltpu.make_async_copy(p_h.at[bk], p_st.at[db], stsem).wait()

        stage(0, 0)

        def pair_body(pr, _):
            for db in range(2):
                bk = pr * 2 + db
                stage_wait(bk, db)
                if db == 0:
                    stage(bk + 1, 1)
                else:
                    @pl.when(pr < NPR2 - 1)
                    def _():
                        stage(bk + 1, 0)

                def vec_body(v, _):
                    sl = pl.ds(v * L, L)
                    sv = s_st[db, sl]
                    dv = d_st[db, sl]
                    pv = p_st[db, sl]
                    hvs = [plsc.load_gather(hts[j], [sv]) for j in range(CPT)]
                    for j in range(CPT):
                        plsc.addupdate_scatter(accs[j], [dv], hvs[j] * pv)
                    return 0
                lax.fori_loop(0, BK2 // L, vec_body, 0, unroll=4)
            return 0
        lax.fori_loop(0, NPR2, pair_body, 0)

        for j in range(CPT):
            pltpu.sync_copy(accs[j], out_h.at[w, j])

    return skb(srcf, dstf, pf, hT)


def _sc_layer(src3, dst3, srcf, dstf, h, a_s, a_d):
    p3, d0, d1 = _sc_weights(src3, dst3, a_s, a_d)
    pf = p3.reshape(NBK2, BK2)
    outB = _sc_aggregate(srcf, dstf, pf, h.T)
    part = outB.reshape(D, NPAD).T
    return part, d0, d1


def kernel(x, edge_index, W1, att_src1, att_dst1, b1, W2, att_src2, att_dst2, b2):
    f32 = jnp.float32
    src = edge_index[0].reshape(NW, EPT)
    dst = edge_index[1].reshape(NW, EPT)
    pad_s = jnp.zeros((NW, EPTP - EPT), jnp.int32)
    pad_d = jnp.full((NW, EPTP - EPT), NPAD - 1, jnp.int32)
    src3 = jnp.concatenate([src, pad_s], axis=1).reshape(NW, NCHUNK, CH)
    dst3 = jnp.concatenate([dst, pad_d], axis=1).reshape(NW, NCHUNK, CH)
    srcf = src3.reshape(NBK2, BK2)
    dstf = dst3.reshape(NBK2, BK2)

    xp = jnp.zeros((NPAD, D), f32).at[:N].set(x)

    h1, a1s, a1d = _tc1(xp, W1, att_src1.reshape(1, D), att_dst1.reshape(1, D))
    part1, d0, d1 = _sc_layer(src3, dst3, srcf, dstf, h1,
                              a1s.reshape(NPAD), a1d.reshape(NPAD))
    h2, a2s, a2d = _tc2(part1, d0.reshape(NPAD, 1), d1.reshape(NPAD, 1),
                        b1.reshape(1, D), W2,
                        att_src2.reshape(1, D), att_dst2.reshape(1, D))
    part2, e0, e1 = _sc_layer(src3, dst3, srcf, dstf, h2,
                              a2s.reshape(NPAD), a2d.reshape(NPAD))
    out = _tc3(part2, e0.reshape(NPAD, 1), e1.reshape(NPAD, 1),
               b2.reshape(1, D))
    return out[:N]
